# Initial kernel scaffold; baseline (speedup 1.0000x reference)
import functools

import jax
import jax.numpy as jnp
import numpy as np
from jax import lax
from jax.experimental import pallas as pl
from jax.experimental.pallas import tpu as pltpu
from jax.experimental.pallas import tpu_sc as plsc

_SHUFFLE_SEED = 42
_B, _N, _D = 4, 4096, 2048
_ROWS = _B * _N
_NC, _NS = 2, 16
_NW = _NC * _NS
_RPW = _ROWS // _NW
_K = 16
_CHUNKS = _RPW // _K

_mesh = plsc.VectorSubcoreMesh(core_axis_name="c", subcore_axis_name="s",
                               num_cores=_NC, num_subcores=_NS)


@functools.partial(
    pl.kernel,
    out_type=jax.ShapeDtypeStruct((_ROWS, _D), jnp.float32),
    mesh=_mesh,
    scratch_types=[
        pltpu.VMEM((_K,), jnp.int32),
        pltpu.VMEM((_K, _D), jnp.float32),
        pltpu.SemaphoreType.DMA,
    ],
)
def _sc_shuffle(table_hbm, idx_hbm, out_hbm, idx_v, rows_v, gsem):
    wid = lax.axis_index("s") * _NC + lax.axis_index("c")
    base = wid * _RPW

    def body(c, carry):
        off = pl.multiple_of(base + c * _K, _K)
        pltpu.sync_copy(idx_hbm.at[pl.ds(off, _K)], idx_v)
        pltpu.async_copy(table_hbm.at[idx_v], rows_v, gsem).wait()
        pltpu.sync_copy(rows_v, out_hbm.at[pl.ds(off, _K)])
        return carry

    lax.fori_loop(0, _CHUNKS, body, 0)


@functools.lru_cache(maxsize=1)
def _global_index() -> np.ndarray:
    perm = np.asarray(jax.random.permutation(jax.random.key(_SHUFFLE_SEED), _N))
    return (np.arange(_B, dtype=np.int64)[:, None] * _N
            + perm[None, :]).astype(np.int32).reshape(-1)


def kernel(inputs):
    flat = inputs.reshape(_ROWS, _D)
    idx = jnp.asarray(_global_index())
    out = _sc_shuffle(flat, idx)
    return out.reshape(_B, _N, _D)


# speedup vs baseline: 2.3848x; 2.3848x over previous
import functools

import jax
import jax.numpy as jnp
import numpy as np
from jax import lax
from jax.experimental import pallas as pl
from jax.experimental.pallas import tpu as pltpu
from jax.experimental.pallas import tpu_sc as plsc

_SHUFFLE_SEED = 42
_B, _N, _D = 4, 4096, 2048
_ROWS = _B * _N
_NC, _NS = 2, 16
_NW = _NC * _NS
_RPW = _ROWS // _NW
_K = 16
_CHUNKS = _RPW // _K

_mesh = plsc.VectorSubcoreMesh(core_axis_name="c", subcore_axis_name="s",
                               num_cores=_NC, num_subcores=_NS)


@functools.partial(
    pl.kernel,
    out_type=jax.ShapeDtypeStruct((_ROWS, _D), jnp.float32),
    mesh=_mesh,
    scratch_types=[
        pltpu.VMEM((_K,), jnp.int32),
        pltpu.VMEM((_K, _D), jnp.float32),
        pltpu.SemaphoreType.DMA,
    ],
)
def _sc_shuffle(table_hbm, idx_hbm, out_hbm, idx_v, rows_v, gsem):
    wid = lax.axis_index("s") * _NC + lax.axis_index("c")
    base = wid * _RPW

    def body(c, carry):
        off = pl.multiple_of(base + c * _K, _K)
        pltpu.sync_copy(idx_hbm.at[pl.ds(off, _K)], idx_v)
        pltpu.async_copy(table_hbm.at[idx_v], rows_v, gsem).wait()
        pltpu.sync_copy(rows_v, out_hbm.at[pl.ds(off, _K)])
        return carry

    lax.fori_loop(0, _CHUNKS, body, 0)


def _traced_global_index():
    perm = jax.random.permutation(jax.random.key(_SHUFFLE_SEED), _N)
    return (jnp.arange(_B, dtype=jnp.int32)[:, None] * _N
            + perm[None, :].astype(jnp.int32)).reshape(-1)


def _eager_global_index():
    with jax.default_device(jax.devices("cpu")[0]):
        return np.asarray(_traced_global_index())


try:
    _GLOBAL_IDX = _eager_global_index()
except Exception:
    _GLOBAL_IDX = None


def kernel(inputs):
    flat = inputs.reshape(_ROWS, _D)
    idx = (jnp.asarray(_GLOBAL_IDX) if _GLOBAL_IDX is not None
           else _traced_global_index())
    out = _sc_shuffle(flat, idx)
    return out.reshape(_B, _N, _D)


# speedup vs baseline: 3.1304x; 1.3126x over previous
import functools

import jax
import jax.numpy as jnp
import numpy as np
from jax import lax
from jax.experimental import pallas as pl
from jax.experimental.pallas import tpu as pltpu
from jax.experimental.pallas import tpu_sc as plsc

_SHUFFLE_SEED = 42
_B, _N, _D = 4, 4096, 2048
_ROWS = _B * _N
_NC, _NS = 2, 16
_NW = _NC * _NS
_RPW = _ROWS // _NW
_K = 16
_CHUNKS = _RPW // _K

_mesh = plsc.VectorSubcoreMesh(core_axis_name="c", subcore_axis_name="s",
                               num_cores=_NC, num_subcores=_NS)


@functools.partial(
    pl.kernel,
    out_type=jax.ShapeDtypeStruct((_ROWS, _D), jnp.float32),
    mesh=_mesh,
    scratch_types=[
        pltpu.VMEM((_RPW,), jnp.int32),
        pltpu.VMEM((_K, _D), jnp.float32),
        pltpu.VMEM((_K, _D), jnp.float32),
        pltpu.SemaphoreType.DMA,
        pltpu.SemaphoreType.DMA,
        pltpu.SemaphoreType.DMA,
        pltpu.SemaphoreType.DMA,
    ],
)
def _sc_shuffle(table_hbm, idx_hbm, out_hbm, idx_v, bufa, bufb,
                gsa, gsb, osa, osb):
    wid = lax.axis_index("s") * _NC + lax.axis_index("c")
    base = wid * _RPW

    pltpu.sync_copy(idx_hbm.at[pl.ds(base, _RPW)], idx_v)

    def gather_desc(g, buf, sem):
        off = pl.multiple_of(g * _K, _K)
        return pltpu.make_async_copy(
            table_hbm.at[idx_v.at[pl.ds(off, _K)]], buf, sem)

    def out_desc(g, buf, sem):
        off = pl.multiple_of(base + g * _K, _K)
        return pltpu.make_async_copy(buf, out_hbm.at[pl.ds(off, _K)], sem)

    gather_desc(0, bufa, gsa).start()
    gather_desc(1, bufb, gsb).start()

    def step(g, buf, gsem, osem):
        gather_desc(g, buf, gsem).wait()
        out_desc(g, buf, osem).start()
        out_desc(g, buf, osem).wait()
        gather_desc(g + 2, buf, gsem).start()

    def body(j, carry):
        g0 = 2 * j
        step(g0, bufa, gsa, osa)
        step(g0 + 1, bufb, gsb, osb)
        return carry

    lax.fori_loop(0, (_CHUNKS - 2) // 2, body, 0)

    gather_desc(_CHUNKS - 2, bufa, gsa).wait()
    out_desc(_CHUNKS - 2, bufa, osa).start()
    gather_desc(_CHUNKS - 1, bufb, gsb).wait()
    out_desc(_CHUNKS - 1, bufb, osb).start()
    out_desc(_CHUNKS - 2, bufa, osa).wait()
    out_desc(_CHUNKS - 1, bufb, osb).wait()


def _traced_global_index():
    perm = jax.random.permutation(jax.random.key(_SHUFFLE_SEED), _N)
    return (jnp.arange(_B, dtype=jnp.int32)[:, None] * _N
            + perm[None, :].astype(jnp.int32)).reshape(-1)


def _eager_global_index():
    with jax.default_device(jax.devices("cpu")[0]):
        return np.asarray(_traced_global_index())


try:
    _GLOBAL_IDX = _eager_global_index()
except Exception:
    _GLOBAL_IDX = None


def kernel(inputs):
    flat = inputs.reshape(_ROWS, _D)
    idx = (jnp.asarray(_GLOBAL_IDX) if _GLOBAL_IDX is not None
           else _traced_global_index())
    out = _sc_shuffle(flat, idx)
    return out.reshape(_B, _N, _D)


# speedup vs baseline: 3.1384x; 1.0026x over previous
import functools

import jax
import jax.numpy as jnp
import numpy as np
from jax import lax
from jax.experimental import pallas as pl
from jax.experimental.pallas import tpu as pltpu
from jax.experimental.pallas import tpu_sc as plsc

_SHUFFLE_SEED = 42
_B, _N, _D = 4, 4096, 2048
_ROWS = _B * _N
_NC, _NS = 2, 16
_NW = _NC * _NS
_RPW = _ROWS // _NW
_K = 8
_CHUNKS = _RPW // _K
_NBUF = 4

_mesh = plsc.VectorSubcoreMesh(core_axis_name="c", subcore_axis_name="s",
                               num_cores=_NC, num_subcores=_NS)


@functools.partial(
    pl.kernel,
    out_type=jax.ShapeDtypeStruct((_ROWS, _D), jnp.float32),
    mesh=_mesh,
    scratch_types=(
        [pltpu.VMEM((_RPW,), jnp.int32)]
        + [pltpu.VMEM((_K, _D), jnp.float32) for _ in range(_NBUF)]
        + [pltpu.SemaphoreType.DMA for _ in range(2 * _NBUF)]
    ),
)
def _sc_shuffle(table_hbm, idx_hbm, out_hbm, idx_v, *bufs_and_sems):
    bufs = bufs_and_sems[:_NBUF]
    gsems = bufs_and_sems[_NBUF:2 * _NBUF]
    osems = bufs_and_sems[2 * _NBUF:]
    wid = lax.axis_index("s") * _NC + lax.axis_index("c")
    base = wid * _RPW

    pltpu.sync_copy(idx_hbm.at[pl.ds(base, _RPW)], idx_v)

    def gather_desc(g, s):
        off = pl.multiple_of(g * _K, _K)
        return pltpu.make_async_copy(
            table_hbm.at[idx_v.at[pl.ds(off, _K)]], bufs[s], gsems[s])

    def out_desc(g, s):
        off = pl.multiple_of(base + g * _K, _K)
        return pltpu.make_async_copy(bufs[s], out_hbm.at[pl.ds(off, _K)],
                                     osems[s])

    gather_desc(0, 0).start()
    gather_desc(1, 1).start()
    for g in (0, 1):
        gather_desc(g + 2, (g + 2) % _NBUF).start()
        gather_desc(g, g % _NBUF).wait()
        out_desc(g, g % _NBUF).start()

    def step(g, s):
        t = (s + 2) % _NBUF
        out_desc(g - 2, t).wait()
        gather_desc(g + 2, t).start()
        gather_desc(g, s).wait()
        out_desc(g, s).start()

    def body(j, carry):
        g0 = 2 + _NBUF * j
        for i in range(_NBUF):
            step(g0 + i, (2 + i) % _NBUF)
        return carry

    lax.fori_loop(0, (_CHUNKS - 4) // _NBUF, body, 0)

    for g in (_CHUNKS - 2, _CHUNKS - 1):
        out_desc(g - 2, (g - 2) % _NBUF).wait()
        gather_desc(g, g % _NBUF).wait()
        out_desc(g, g % _NBUF).start()
    for g in (_CHUNKS - 2, _CHUNKS - 1):
        out_desc(g, g % _NBUF).wait()


def _traced_global_index():
    perm = jax.random.permutation(jax.random.key(_SHUFFLE_SEED), _N)
    return (jnp.arange(_B, dtype=jnp.int32)[:, None] * _N
            + perm[None, :].astype(jnp.int32)).reshape(-1)


def _eager_global_index():
    with jax.default_device(jax.devices("cpu")[0]):
        return np.asarray(_traced_global_index())


try:
    _GLOBAL_IDX = _eager_global_index()
except Exception:
    _GLOBAL_IDX = None


def kernel(inputs):
    flat = inputs.reshape(_ROWS, _D)
    idx = (jnp.asarray(_GLOBAL_IDX) if _GLOBAL_IDX is not None
           else _traced_global_index())
    out = _sc_shuffle(flat, idx)
    return out.reshape(_B, _N, _D)
